# baseline (speedup 1.0000x reference)
import functools

import jax
import jax.numpy as jnp
from jax import lax
from jax.experimental import pallas as pl
from jax.experimental.pallas import tpu as pltpu
from jax.experimental.pallas import tpu_sc as plsc

_I = 13
_LB = -500000.0
_N_STATES = 1000001

_B = 16384
_NC = 2
_NS = 16
_NW = _NC * _NS
_BPW = _B // _NW
_CHUNK = 128
_NCHUNK = _BPW // _CHUNK
_LANES = 16
_MAGIC = 8388608.0

_STAGE = 62496
_TAIL_OFF = 16 * _STAGE
_TAIL = _N_STATES - _TAIL_OFF

_mesh = plsc.VectorSubcoreMesh(core_axis_name="c", subcore_axis_name="s")


@functools.partial(
    pl.kernel,
    mesh=_mesh,
    out_type=jax.ShapeDtypeStruct((_B,), jnp.float32),
    scratch_types=[
        pltpu.VMEM((_BPW,), jnp.float32),
        pltpu.VMEM((_NCHUNK, _CHUNK), jnp.int32),
        pltpu.VMEM((_BPW,), jnp.float32),
        pltpu.VMEM((_STAGE + _TAIL,), jnp.float32),
        pltpu.VMEM_SHARED((_N_STATES,), jnp.float32),
        pltpu.SemaphoreType.DMA,
        pltpu.SemaphoreType.DMA,
    ],
)
def _table_gather(xi_hbm, table_hbm, out_hbm, xi_v, idx_v, val_v, bounce_v, tab_s,
                  sem, ssem):
    cid = lax.axis_index("c")
    sid = lax.axis_index("s")
    wid = sid * _NC + cid
    base = wid * _BPW

    soff = sid * _STAGE
    stage = pltpu.async_copy(
        table_hbm.at[pl.ds(soff, _STAGE)], bounce_v.at[pl.ds(0, _STAGE)], ssem)

    @pl.when(sid == 0)
    def _():
        pltpu.sync_copy(table_hbm.at[pl.ds(_TAIL_OFF, _TAIL)],
                        bounce_v.at[pl.ds(_STAGE, _TAIL)])
        pltpu.sync_copy(bounce_v.at[pl.ds(_STAGE, _TAIL)],
                        tab_s.at[pl.ds(_TAIL_OFF, _TAIL)])

    pltpu.sync_copy(xi_hbm.at[pl.ds(base, _BPW)], xi_v)

    groups_per_chunk = _CHUNK // _LANES
    for j in range(_BPW // _LANES):
        xi = xi_v[pl.ds(j * _LANES, _LANES)]
        v = xi - _LB
        v = jnp.minimum(jnp.maximum(v, 0.0), float(_N_STATES - 1))
        v = (v + _MAGIC) - _MAGIC
        idx = v.astype(jnp.int32)
        idx_v[j // groups_per_chunk,
              pl.ds((j % groups_per_chunk) * _LANES, _LANES)] = idx

    stage.wait()
    pltpu.sync_copy(bounce_v.at[pl.ds(0, _STAGE)], tab_s.at[pl.ds(soff, _STAGE)])
    plsc.subcore_barrier()

    for c in range(_NCHUNK):
        pltpu.async_copy(
            tab_s.at[idx_v.at[c]],
            val_v.at[pl.ds(c * _CHUNK, _CHUNK)],
            sem,
        ).wait()

    pltpu.sync_copy(val_v, out_hbm.at[pl.ds(base, _BPW)])


def kernel(x, table):
    return _table_gather(lax.slice(x, (0, _I), (_B, _I + 1)).reshape(_B), table)[:, None]
